# Initial kernel scaffold; baseline (speedup 1.0000x reference)
import jax
import jax.numpy as jnp
from jax.experimental import pallas as pl


def kernel(x, edge_index, sig_conv1_w, sig_conv1_b, fc1_w, fc1_b, fc2_w, fc2_b, fc3_w, fc3_b, fc4_w, fc4_b, conv1_w, conv1_b, conv2_w, conv2_b):
    raise NotImplementedError("write your pallas kernel here")


# speedup vs baseline: 21.1510x; 21.1510x over previous
import functools

import jax
import jax.numpy as jnp
from jax import lax
from jax.experimental import pallas as pl
from jax.experimental.pallas import tpu as pltpu
from jax.experimental.pallas import tpu_sc as plsc

N_NODES = 10000
N_EDGES = 320000
D_IN = 128
D_OUT = 32
D_HID = 64

NC = 2
NS = 16
NW = NC * NS
CHUNK = 128
NCH = -(-N_EDGES // (NW * CHUNK))
E_PAD = NW * NCH * CHUNK
N_PAD = 10112
ROWS_PER_TILE = N_PAD // NS
DEG_W = 16

_MESH = plsc.VectorSubcoreMesh(core_axis_name="c", subcore_axis_name="s")


def _fill(buf, rows, width, value):
    v = jnp.full((16,), value, jnp.float32)

    def body(r, carry):
        for k in range(width // 16):
            buf[r, pl.ds(k * 16, 16)] = v
        return carry

    lax.fori_loop(0, rows, body, 0)


def _zero_acc_slice(zbuf, acc, sid, width):
    base = sid * ROWS_PER_TILE
    full = ROWS_PER_TILE // CHUNK
    rem = ROWS_PER_TILE % CHUNK
    for k in range(full):
        pltpu.sync_copy(zbuf, acc.at[pl.ds(base + k * CHUNK, CHUNK)])
    if rem:
        pltpu.sync_copy(zbuf.at[pl.ds(0, rem)],
                        acc.at[pl.ds(base + full * CHUNK, rem)])


@functools.partial(
    pl.kernel,
    out_type=jax.ShapeDtypeStruct((NC, N_PAD, DEG_W), jnp.float32),
    mesh=_MESH,
    compiler_params=pltpu.CompilerParams(use_tc_tiling_on_sc=False),
    scratch_types=[
        pltpu.VMEM((NCH, CHUNK), jnp.int32),
        pltpu.VMEM((CHUNK, DEG_W), jnp.float32),
        pltpu.VMEM_SHARED((N_PAD, DEG_W), jnp.float32),
    ],
)
def _deg_kernel(row_hbm, out_hbm, row_v, buf, acc):
    c = lax.axis_index("c")
    s = lax.axis_index("s")
    wid = c * NS + s
    pltpu.sync_copy(row_hbm.at[wid], row_v)
    _fill(buf, CHUNK, DEG_W, 0.0)
    _zero_acc_slice(buf, acc, s, DEG_W)
    plsc.subcore_barrier()
    _fill(buf, CHUNK, DEG_W, 1.0)

    def body(j, carry):
        pltpu.sync_copy(buf, acc.at[row_v.at[j]], add=True)
        return carry

    lax.fori_loop(0, NCH, body, 0)
    plsc.subcore_barrier()
    sl = pl.ds(s * ROWS_PER_TILE, ROWS_PER_TILE)
    pltpu.sync_copy(acc.at[sl], out_hbm.at[c, sl])


def _make_agg(D):
    params = (None if D % 128 == 0
              else pltpu.CompilerParams(use_tc_tiling_on_sc=False))

    @functools.partial(
        pl.kernel,
        out_type=jax.ShapeDtypeStruct((NC, N_PAD, D), jnp.float32),
        mesh=_MESH,
        compiler_params=params,
        scratch_types=[
            pltpu.VMEM((NCH, CHUNK), jnp.int32),
            pltpu.VMEM((NCH, CHUNK), jnp.int32),
            pltpu.VMEM((CHUNK, D), jnp.float32),
            pltpu.VMEM_SHARED((N_PAD, D), jnp.float32),
            pltpu.SemaphoreType.DMA,
        ],
    )
    def agg_kernel(y_hbm, row_hbm, col_hbm, out_hbm, row_v, col_v, buf, acc, sem):
        c = lax.axis_index("c")
        s = lax.axis_index("s")
        wid = c * NS + s
        pltpu.sync_copy(row_hbm.at[wid], row_v)
        pltpu.sync_copy(col_hbm.at[wid], col_v)
        _fill(buf, CHUNK, D, 0.0)
        _zero_acc_slice(buf, acc, s, D)
        plsc.subcore_barrier()

        def body(j, carry):
            pltpu.async_copy(y_hbm.at[col_v.at[j]], buf, sem).wait()
            pltpu.sync_copy(buf, acc.at[row_v.at[j]], add=True)
            return carry

        lax.fori_loop(0, NCH, body, 0)
        plsc.subcore_barrier()
        sl = pl.ds(s * ROWS_PER_TILE, ROWS_PER_TILE)
        pltpu.sync_copy(acc.at[sl], out_hbm.at[c, sl])

    return agg_kernel


_agg128 = _make_agg(D_IN)
_agg32 = _make_agg(D_OUT)

_BR = 2000
_NB = N_NODES // _BR


def _scale_body(degp_ref, x_ref, wcat_ref, y_ref, dinv_ref):
    d = degp_ref[...]
    cnt = d[0, :, 0:1] + d[1, :, 0:1] + 1.0
    dinv = lax.rsqrt(jnp.clip(cnt, 1e-12, None))
    dinv_ref[...] = dinv
    xw = jnp.dot(x_ref[...], wcat_ref[...], preferred_element_type=jnp.float32)
    y_ref[...] = xw * dinv


def _scale(degp, x, wcat):
    return pl.pallas_call(
        _scale_body,
        grid=(_NB,),
        in_specs=[
            pl.BlockSpec((NC, _BR, DEG_W), lambda i: (0, i, 0)),
            pl.BlockSpec((_BR, D_IN), lambda i: (i, 0)),
            pl.BlockSpec((D_IN, 2 * D_HID), lambda i: (0, 0)),
        ],
        out_specs=[
            pl.BlockSpec((_BR, 2 * D_HID), lambda i: (i, 0)),
            pl.BlockSpec((_BR, 1), lambda i: (i, 0)),
        ],
        out_shape=[
            jax.ShapeDtypeStruct((N_NODES, 2 * D_HID), jnp.float32),
            jax.ShapeDtypeStruct((N_NODES, 1), jnp.float32),
        ],
    )(degp, x, wcat)


def _dense1_body(p_ref, y_ref, dinv_ref, sigb_ref, h1pre_ref, s_ref):
    i = pl.program_id(0)
    p = p_ref[...]
    a = (p[0] + p[1] + y_ref[...]) * dinv_ref[...]
    h = jnp.maximum(a[:, :D_HID] + sigb_ref[...], 0.0)
    h1pre_ref[...] = a[:, D_HID:]
    part = jnp.sum(h, axis=0, keepdims=True)

    @pl.when(i == 0)
    def _():
        s_ref[...] = part

    @pl.when(i > 0)
    def _():
        s_ref[...] += part


def _dense1(p, y, dinv, sigb):
    return pl.pallas_call(
        _dense1_body,
        grid=(_NB,),
        in_specs=[
            pl.BlockSpec((NC, _BR, 2 * D_HID), lambda i: (0, i, 0)),
            pl.BlockSpec((_BR, 2 * D_HID), lambda i: (i, 0)),
            pl.BlockSpec((_BR, 1), lambda i: (i, 0)),
            pl.BlockSpec((1, D_HID), lambda i: (0, 0)),
        ],
        out_specs=[
            pl.BlockSpec((_BR, D_HID), lambda i: (i, 0)),
            pl.BlockSpec((1, D_HID), lambda i: (0, 0)),
        ],
        out_shape=[
            jax.ShapeDtypeStruct((N_NODES, D_HID), jnp.float32),
            jax.ShapeDtypeStruct((1, D_HID), jnp.float32),
        ],
    )(p, y, dinv, sigb)


def _dense2_body(s_ref, f1t_ref, f1b_ref, f2t_ref, f2b_ref, f3t_ref, f3b_ref,
                 f4t_ref, f4b_ref, h1pre_ref, dinv_ref, c1b_ref, w2_ref,
                 c2b_ref, y2_ref, g2_ref, b2c_ref):
    s = s_ref[...]
    g1 = jnp.tanh(jnp.dot(s, f1t_ref[...], preferred_element_type=jnp.float32)
                  + f1b_ref[...])
    be1 = jnp.tanh(jnp.dot(s, f2t_ref[...], preferred_element_type=jnp.float32)
                   + f2b_ref[...])
    g2 = jnp.tanh(jnp.dot(s, f3t_ref[...], preferred_element_type=jnp.float32)
                  + f3b_ref[...])
    be2 = jnp.tanh(jnp.dot(s, f4t_ref[...], preferred_element_type=jnp.float32)
                   + f4b_ref[...])
    h1 = jnp.maximum(g1 * h1pre_ref[...] + be1 + c1b_ref[...], 0.0)
    y2_ref[...] = dinv_ref[...] * jnp.dot(
        h1, w2_ref[...], preferred_element_type=jnp.float32)
    g2_ref[...] = g2
    b2c_ref[...] = be2 + c2b_ref[...]


def _dense2(s, f1t, f1b, f2t, f2b, f3t, f3b, f4t, f4b, h1pre, dinv, c1b, w2, c2b):
    small = lambda r, c: pl.BlockSpec((r, c), lambda i: (0, 0))
    return pl.pallas_call(
        _dense2_body,
        grid=(_NB,),
        in_specs=[
            small(1, D_HID),
            small(D_HID, D_HID), small(1, D_HID),
            small(D_HID, D_HID), small(1, D_HID),
            small(D_HID, D_OUT), small(1, D_OUT),
            small(D_HID, D_OUT), small(1, D_OUT),
            pl.BlockSpec((_BR, D_HID), lambda i: (i, 0)),
            pl.BlockSpec((_BR, 1), lambda i: (i, 0)),
            small(1, D_HID),
            small(D_HID, D_OUT),
            small(1, D_OUT),
        ],
        out_specs=[
            pl.BlockSpec((_BR, D_OUT), lambda i: (i, 0)),
            small(1, D_OUT),
            small(1, D_OUT),
        ],
        out_shape=[
            jax.ShapeDtypeStruct((N_NODES, D_OUT), jnp.float32),
            jax.ShapeDtypeStruct((1, D_OUT), jnp.float32),
            jax.ShapeDtypeStruct((1, D_OUT), jnp.float32),
        ],
    )(s, f1t, f1b, f2t, f2b, f3t, f3b, f4t, f4b, h1pre, dinv, c1b, w2, c2b)


def _epilogue_body(q_ref, y2_ref, dinv_ref, g2_ref, b2c_ref, out_ref):
    q = q_ref[...]
    agg2 = (q[0] + q[1] + y2_ref[...]) * dinv_ref[...]
    out_ref[...] = g2_ref[...] * agg2 + b2c_ref[...]


def _epilogue(q, y2, dinv, g2, b2c):
    return pl.pallas_call(
        _epilogue_body,
        grid=(1,),
        in_specs=[
            pl.BlockSpec((NC, N_NODES, D_OUT), lambda i: (0, 0, 0)),
            pl.BlockSpec((N_NODES, D_OUT), lambda i: (0, 0)),
            pl.BlockSpec((N_NODES, 1), lambda i: (0, 0)),
            pl.BlockSpec((1, D_OUT), lambda i: (0, 0)),
            pl.BlockSpec((1, D_OUT), lambda i: (0, 0)),
        ],
        out_specs=pl.BlockSpec((N_NODES, D_OUT), lambda i: (0, 0)),
        out_shape=jax.ShapeDtypeStruct((N_NODES, D_OUT), jnp.float32),
    )(q, y2, dinv, g2, b2c)


def kernel(x, edge_index, sig_conv1_w, sig_conv1_b, fc1_w, fc1_b, fc2_w, fc2_b,
           fc3_w, fc3_b, fc4_w, fc4_b, conv1_w, conv1_b, conv2_w, conv2_b):
    row = edge_index[0]
    col = edge_index[1]
    pad = E_PAD - N_EDGES
    rows_p = jnp.concatenate(
        [row, jnp.full((pad,), N_NODES, row.dtype)]).reshape(NW, NCH, CHUNK)
    cols_p = jnp.concatenate(
        [col, jnp.zeros((pad,), col.dtype)]).reshape(NW, NCH, CHUNK)

    degp = _deg_kernel(rows_p)
    wcat = jnp.concatenate([sig_conv1_w, conv1_w], axis=1)
    y, dinv = _scale(degp, x, wcat)
    p = _agg128(y, rows_p, cols_p)
    h1pre, s = _dense1(p, y, dinv, sig_conv1_b.reshape(1, -1))
    y2, g2, b2c = _dense2(
        s, fc1_w.T, fc1_b.reshape(1, -1), fc2_w.T, fc2_b.reshape(1, -1),
        fc3_w.T, fc3_b.reshape(1, -1), fc4_w.T, fc4_b.reshape(1, -1),
        h1pre, dinv, conv1_b.reshape(1, -1), conv2_w, conv2_b.reshape(1, -1))
    q = _agg32(y2, rows_p, cols_p)
    return _epilogue(q, y2, dinv, g2, b2c)
